# baseline (speedup 1.0000x reference)
import jax
import jax.numpy as jnp
import numpy as np
from jax.experimental import pallas as pl
from jax.experimental.pallas import tpu as pltpu

_SIZES = (18, 17, 13, 13, 13, 11, 4)
_OFFS = tuple(int(x) for x in np.cumsum((0,) + _SIZES))
_NK = 7
_H = 256
_D = 768
_TW = 128
_BBLK = 4096

_PAUG = np.zeros((8, _TW), np.float32)
for _k in range(_NK):
    _PAUG[_k, _OFFS[_k]:_OFFS[_k + 1]] = 1.0
    _PAUG[7, _OFFS[_k]:_OFFS[_k + 1]] = _OFFS[_k]


def _body(idx_ref, embcat_ref, w1_ref, b1_ref, w2_ref, b2_ref, paug_ref,
          out_ref, m_ref, w2b_ref):
    @pl.when(pl.program_id(0) == 0)
    def _():
        riota = jax.lax.broadcasted_iota(jnp.int32, (_TW, _H), 0)
        acc = jnp.zeros((_TW, _H), jnp.float32)
        for k in range(_NK):
            mask = (riota >= _OFFS[k]) & (riota < _OFFS[k + 1])
            ek = jnp.where(mask, embcat_ref[...], 0.0)
            acc += jnp.dot(ek, w1_ref[k * _H:(k + 1) * _H, :],
                           preferred_element_type=jnp.float32)
        m_ref[...] = acc.astype(jnp.bfloat16)
        w2b_ref[...] = w2_ref[...].astype(jnp.bfloat16)

    bblk = idx_ref.shape[1]
    ids_t = idx_ref[...].T
    c = jnp.dot(ids_t, paug_ref[...],
                preferred_element_type=jnp.float32).astype(jnp.int32)
    iota = jax.lax.broadcasted_iota(jnp.int32, (bblk, _TW), 1)
    mh = (c == iota).astype(jnp.bfloat16)
    h = jnp.dot(mh, m_ref[...], preferred_element_type=jnp.float32)
    h = jnp.maximum(h + b1_ref[...], 0.0).astype(jnp.bfloat16)
    out_ref[...] = jnp.dot(h, w2b_ref[...],
                           preferred_element_type=jnp.float32) + b2_ref[...]


@jax.jit
def _run(idx8, embcat, W1, b1, W2, b2):
    B = idx8.shape[1]
    grid = B // _BBLK
    return pl.pallas_call(
        _body,
        grid=(grid,),
        in_specs=[
            pl.BlockSpec((8, _BBLK), lambda i: (0, i)),
            pl.BlockSpec((_TW, _H), lambda i: (0, 0)),
            pl.BlockSpec((_H * _NK, _H), lambda i: (0, 0)),
            pl.BlockSpec((1, _H), lambda i: (0, 0)),
            pl.BlockSpec((_H, _D), lambda i: (0, 0)),
            pl.BlockSpec((1, _D), lambda i: (0, 0)),
            pl.BlockSpec((8, _TW), lambda i: (0, 0)),
        ],
        out_specs=pl.BlockSpec((_BBLK, _D), lambda i: (i, 0)),
        out_shape=jax.ShapeDtypeStruct((B, _D), jnp.float32),
        scratch_shapes=[pltpu.VMEM((_TW, _H), jnp.bfloat16),
                        pltpu.VMEM((_H, _D), jnp.bfloat16)],
        compiler_params=pltpu.CompilerParams(
            dimension_semantics=("arbitrary",)),
    )(idx8, embcat, W1, b1, W2, b2, jnp.asarray(_PAUG))


def kernel(idx_primary_color, idx_secondary_color, idx_primary_material,
           idx_secondary_material, idx_style, idx_shape, idx_assembly,
           emb_primary_color, emb_secondary_color, emb_primary_material,
           emb_secondary_material, emb_style, emb_shape, emb_assembly,
           W1, b1, W2, b2):
    idxs = [idx_primary_color, idx_secondary_color, idx_primary_material,
            idx_secondary_material, idx_style, idx_shape, idx_assembly]
    embs = [emb_primary_color, emb_secondary_color, emb_primary_material,
            emb_secondary_material, emb_style, emb_shape, emb_assembly]
    B = idxs[0].shape[0]
    idx8 = jnp.stack([i.astype(jnp.float32) for i in idxs]
                     + [jnp.ones((B,), jnp.float32)], axis=0)
    embcat = jnp.concatenate(
        [e.astype(jnp.float32) for e in embs]
        + [jnp.zeros((_TW - _OFFS[-1], _H), jnp.float32)], axis=0)
    return _run(idx8, embcat, W1, b1.reshape(1, _H), W2, b2.reshape(1, _D))


# speedup vs baseline: 1.8503x; 1.8503x over previous
import jax
import jax.numpy as jnp
import numpy as np
from jax.experimental import pallas as pl
from jax.experimental.pallas import tpu as pltpu

_SIZES = (18, 17, 13, 13, 13, 11, 4)
_OFFS = tuple(int(x) for x in np.cumsum((0,) + _SIZES))
_NK = 7
_H = 256
_D = 768
_TW = 128
_BBLK = 4096

_PAUG = np.zeros((8, _TW), np.float32)
for _k in range(_NK):
    _PAUG[_k, _OFFS[_k]:_OFFS[_k + 1]] = 1.0
    _PAUG[7, _OFFS[_k]:_OFFS[_k + 1]] = _OFFS[_k]


def _body(i0, i1, i2, i3, i4, i5, i6, e0, e1, e2, e3, e4, e5, e6,
          w1_ref, b1_ref, w2_ref, b2_ref, paug_ref,
          out_ref, m_ref, w2b_ref):
    @pl.when(pl.program_id(0) == 0)
    def _():
        embcat = jnp.concatenate(
            [e[...] for e in (e0, e1, e2, e3, e4, e5, e6)]
            + [jnp.zeros((_TW - _OFFS[-1], _H), jnp.float32)], axis=0)
        riota = jax.lax.broadcasted_iota(jnp.int32, (_TW, _H), 0)
        acc = jnp.zeros((_TW, _H), jnp.float32)
        for k in range(_NK):
            mask = (riota >= _OFFS[k]) & (riota < _OFFS[k + 1])
            ek = jnp.where(mask, embcat, 0.0)
            acc += jnp.dot(ek, w1_ref[k * _H:(k + 1) * _H, :],
                           preferred_element_type=jnp.float32)
        m_ref[...] = acc.astype(jnp.bfloat16)
        w2b_ref[...] = w2_ref[...].astype(jnp.bfloat16)

    bblk = i0.shape[0]
    ids8 = jnp.stack(
        [r[...].astype(jnp.float32) for r in (i0, i1, i2, i3, i4, i5, i6)]
        + [jnp.ones((bblk,), jnp.float32)], axis=0)
    ids_t = ids8.T
    c = jnp.dot(ids_t, paug_ref[...],
                preferred_element_type=jnp.float32).astype(jnp.int32)
    iota = jax.lax.broadcasted_iota(jnp.int32, (bblk, _TW), 1)
    mh = (c == iota).astype(jnp.bfloat16)
    h = jnp.dot(mh, m_ref[...], preferred_element_type=jnp.float32)
    h = jnp.maximum(h + b1_ref[...], 0.0).astype(jnp.bfloat16)
    out_ref[...] = jnp.dot(h, w2b_ref[...],
                           preferred_element_type=jnp.float32) + b2_ref[...]


@jax.jit
def _run(idxs, embs, W1, b1, W2, b2):
    B = idxs[0].shape[0]
    grid = B // _BBLK
    idx_specs = [pl.BlockSpec((_BBLK,), lambda i: (i,)) for _ in range(_NK)]
    emb_specs = [pl.BlockSpec((_SIZES[k], _H), lambda i: (0, 0))
                 for k in range(_NK)]
    return pl.pallas_call(
        _body,
        grid=(grid,),
        in_specs=idx_specs + emb_specs + [
            pl.BlockSpec((_H * _NK, _H), lambda i: (0, 0)),
            pl.BlockSpec((1, _H), lambda i: (0, 0)),
            pl.BlockSpec((_H, _D), lambda i: (0, 0)),
            pl.BlockSpec((1, _D), lambda i: (0, 0)),
            pl.BlockSpec((8, _TW), lambda i: (0, 0)),
        ],
        out_specs=pl.BlockSpec((_BBLK, _D), lambda i: (i, 0)),
        out_shape=jax.ShapeDtypeStruct((B, _D), jnp.float32),
        scratch_shapes=[pltpu.VMEM((_TW, _H), jnp.bfloat16),
                        pltpu.VMEM((_H, _D), jnp.bfloat16)],
        compiler_params=pltpu.CompilerParams(
            dimension_semantics=("arbitrary",)),
    )(*idxs, *embs, W1, b1, W2, b2, jnp.asarray(_PAUG))


def kernel(idx_primary_color, idx_secondary_color, idx_primary_material,
           idx_secondary_material, idx_style, idx_shape, idx_assembly,
           emb_primary_color, emb_secondary_color, emb_primary_material,
           emb_secondary_material, emb_style, emb_shape, emb_assembly,
           W1, b1, W2, b2):
    idxs = [idx_primary_color.astype(jnp.int32),
            idx_secondary_color.astype(jnp.int32),
            idx_primary_material.astype(jnp.int32),
            idx_secondary_material.astype(jnp.int32),
            idx_style.astype(jnp.int32),
            idx_shape.astype(jnp.int32),
            idx_assembly.astype(jnp.int32)]
    embs = [emb_primary_color, emb_secondary_color, emb_primary_material,
            emb_secondary_material, emb_style, emb_shape, emb_assembly]
    return _run(idxs, embs, W1, b1.reshape(1, _H), W2, b2.reshape(1, _D))


# speedup vs baseline: 1.8723x; 1.0119x over previous
import jax
import jax.numpy as jnp
import numpy as np
from jax.experimental import pallas as pl
from jax.experimental.pallas import tpu as pltpu

_SIZES = (18, 17, 13, 13, 13, 11, 4)
_OFFS = tuple(int(x) for x in np.cumsum((0,) + _SIZES))
_NK = 7
_H = 256
_D = 768
_TW = 128
_BBLK = 2048

_PAUG = np.zeros((8, _TW), np.float32)
for _k in range(_NK):
    _PAUG[_k, _OFFS[_k]:_OFFS[_k + 1]] = 1.0
    _PAUG[7, _OFFS[_k]:_OFFS[_k + 1]] = _OFFS[_k]


def _body(i0, i1, i2, i3, i4, i5, i6, e0, e1, e2, e3, e4, e5, e6,
          w1_ref, b1_ref, w2_ref, b2_ref, paug_ref,
          out_ref, m_ref, w2b_ref):
    @pl.when(pl.program_id(0) == 0)
    def _():
        embcat = jnp.concatenate(
            [e[...] for e in (e0, e1, e2, e3, e4, e5, e6)]
            + [jnp.zeros((_TW - _OFFS[-1], _H), jnp.float32)], axis=0)
        riota = jax.lax.broadcasted_iota(jnp.int32, (_TW, _H), 0)
        acc = jnp.zeros((_TW, _H), jnp.float32)
        for k in range(_NK):
            mask = (riota >= _OFFS[k]) & (riota < _OFFS[k + 1])
            ek = jnp.where(mask, embcat, 0.0)
            acc += jnp.dot(ek, w1_ref[k * _H:(k + 1) * _H, :],
                           preferred_element_type=jnp.float32)
        m_ref[...] = acc.astype(jnp.bfloat16)
        w2b_ref[...] = w2_ref[...].astype(jnp.bfloat16)

    bblk = i0.shape[0]
    ids8 = jnp.stack(
        [r[...].astype(jnp.float32) for r in (i0, i1, i2, i3, i4, i5, i6)]
        + [jnp.ones((bblk,), jnp.float32)], axis=0)
    ids_t = ids8.T
    c = jnp.dot(ids_t, paug_ref[...],
                preferred_element_type=jnp.float32).astype(jnp.int32)
    iota = jax.lax.broadcasted_iota(jnp.int32, (bblk, _TW), 1)
    mh = (c == iota).astype(jnp.bfloat16)
    h = jnp.dot(mh, m_ref[...], preferred_element_type=jnp.float32)
    h = jnp.maximum(h + b1_ref[...], 0.0).astype(jnp.bfloat16)
    out_ref[...] = jnp.dot(h, w2b_ref[...],
                           preferred_element_type=jnp.float32) + b2_ref[...]


@jax.jit
def _run(idxs, embs, W1, b1, W2, b2):
    B = idxs[0].shape[0]
    grid = B // _BBLK
    idx_specs = [pl.BlockSpec((_BBLK,), lambda i: (i,)) for _ in range(_NK)]
    emb_specs = [pl.BlockSpec((_SIZES[k], _H), lambda i: (0, 0))
                 for k in range(_NK)]
    return pl.pallas_call(
        _body,
        grid=(grid,),
        in_specs=idx_specs + emb_specs + [
            pl.BlockSpec((_H * _NK, _H), lambda i: (0, 0)),
            pl.BlockSpec((1, _H), lambda i: (0, 0)),
            pl.BlockSpec((_H, _D), lambda i: (0, 0)),
            pl.BlockSpec((1, _D), lambda i: (0, 0)),
            pl.BlockSpec((8, _TW), lambda i: (0, 0)),
        ],
        out_specs=pl.BlockSpec((_BBLK, _D), lambda i: (i, 0)),
        out_shape=jax.ShapeDtypeStruct((B, _D), jnp.float32),
        scratch_shapes=[pltpu.VMEM((_TW, _H), jnp.bfloat16),
                        pltpu.VMEM((_H, _D), jnp.bfloat16)],
        compiler_params=pltpu.CompilerParams(
            dimension_semantics=("arbitrary",)),
    )(*idxs, *embs, W1, b1, W2, b2, jnp.asarray(_PAUG))


def kernel(idx_primary_color, idx_secondary_color, idx_primary_material,
           idx_secondary_material, idx_style, idx_shape, idx_assembly,
           emb_primary_color, emb_secondary_color, emb_primary_material,
           emb_secondary_material, emb_style, emb_shape, emb_assembly,
           W1, b1, W2, b2):
    idxs = [idx_primary_color.astype(jnp.int32),
            idx_secondary_color.astype(jnp.int32),
            idx_primary_material.astype(jnp.int32),
            idx_secondary_material.astype(jnp.int32),
            idx_style.astype(jnp.int32),
            idx_shape.astype(jnp.int32),
            idx_assembly.astype(jnp.int32)]
    embs = [emb_primary_color, emb_secondary_color, emb_primary_material,
            emb_secondary_material, emb_style, emb_shape, emb_assembly]
    return _run(idxs, embs, W1, b1.reshape(1, _H), W2, b2.reshape(1, _D))
